# baseline (speedup 1.0000x reference)
import functools

import jax
import jax.numpy as jnp
from jax import lax
from jax.experimental import pallas as pl
from jax.experimental.pallas import tpu as pltpu
from jax.experimental.pallas import tpu_sc as plsc

NC = 2
NS = 16
NW = NC * NS
NBUF = 2


@functools.partial(jax.jit, static_argnames=("chunk",))
def _sc_gather(edge_types, table, chunk):
    B = edge_types.shape[0]
    D = table.shape[1]
    b_per_w = B // NW
    n_chunks = b_per_w // chunk
    n_super = n_chunks // NBUF

    mesh = plsc.VectorSubcoreMesh(
        core_axis_name="c", subcore_axis_name="s", num_cores=NC, num_subcores=NS
    )

    @functools.partial(
        pl.kernel,
        out_type=jax.ShapeDtypeStruct((B, D), jnp.float32),
        mesh=mesh,
        scratch_types=[
            pltpu.VMEM((chunk,), jnp.int32),
            pltpu.VMEM((chunk,), jnp.int32),
            pltpu.VMEM((chunk, D), jnp.float32),
            pltpu.VMEM((chunk, D), jnp.float32),
            pltpu.SemaphoreType.DMA,
            pltpu.SemaphoreType.DMA,
            pltpu.SemaphoreType.DMA,
            pltpu.SemaphoreType.DMA,
            pltpu.SemaphoreType.DMA,
            pltpu.SemaphoreType.DMA,
        ],
        compiler_params=pltpu.CompilerParams(use_tc_tiling_on_sc=False),
    )
    def k(idx_hbm, table_hbm, out_hbm, i0, i1, r0, r1, si0, si1, sg0, sg1, sw0, sw1):
        wid = lax.axis_index("s") * NC + lax.axis_index("c")
        base = wid * b_per_w
        idx_v, rows_v = [i0, i1], [r0, r1]
        sem_i, sem_g, sem_w = [si0, si1], [sg0, sg1], [sw0, sw1]

        def start_idx(g, b):
            pltpu.make_async_copy(
                idx_hbm.at[pl.ds(base + g * chunk, chunk)], idx_v[b], sem_i[b]
            ).start()

        def wait_idx(b):
            pltpu.make_async_copy(
                idx_hbm.at[pl.ds(base, chunk)], idx_v[b], sem_i[b]
            ).wait()

        def start_gather(b):
            pltpu.make_async_copy(
                table_hbm.at[idx_v[b]], rows_v[b], sem_g[b]
            ).start()

        def wait_gather(b):
            pltpu.make_async_copy(
                table_hbm.at[idx_v[b]], rows_v[b], sem_g[b]
            ).wait()

        def start_wb(g, b):
            pltpu.make_async_copy(
                rows_v[b], out_hbm.at[pl.ds(base + g * chunk, chunk)], sem_w[b]
            ).start()

        def wait_wb(b):
            pltpu.make_async_copy(
                rows_v[b], out_hbm.at[pl.ds(base, chunk)], sem_w[b]
            ).wait()

        for b in range(NBUF):
            start_idx(b, b)
        for b in range(NBUF):
            wait_idx(b)
            start_gather(b)

        def body(s, _):
            g0 = s * NBUF
            for b in range(NBUF):
                wait_gather(b)
                start_idx(g0 + NBUF + b, b)
                start_wb(g0 + b, b)
            for b in range(NBUF):
                wait_idx(b)
                wait_wb(b)
                start_gather(b)
            return 0

        lax.fori_loop(0, n_super - 1, body, 0)

        for b in range(NBUF):
            wait_gather(b)
            start_wb(n_chunks - NBUF + b, b)
        for b in range(NBUF):
            wait_wb(b)

    return k(edge_types, table)


def kernel(edge_types, table):
    return _sc_gather(edge_types.astype(jnp.int32), table, 1000)


# speedup vs baseline: 1.2597x; 1.2597x over previous
import functools

import jax
import jax.numpy as jnp
from jax import lax
from jax.experimental import pallas as pl
from jax.experimental.pallas import tpu as pltpu
from jax.experimental.pallas import tpu_sc as plsc

NC = 2
NS = 16
NW = NC * NS
NBUF = 2


@functools.partial(jax.jit, static_argnames=("chunk",))
def _sc_gather(edge_types, table, chunk):
    B = edge_types.shape[0]
    V, D = table.shape
    b_per_w = B // NW
    n_chunks = b_per_w // chunk
    n_super = n_chunks // NBUF

    mesh = plsc.VectorSubcoreMesh(
        core_axis_name="c", subcore_axis_name="s", num_cores=NC, num_subcores=NS
    )

    @functools.partial(
        pl.kernel,
        out_type=jax.ShapeDtypeStruct((B, D), jnp.float32),
        mesh=mesh,
        scratch_types=[
            pltpu.VMEM_SHARED((V, D), jnp.float32),
            pltpu.VMEM((chunk,), jnp.int32),
            pltpu.VMEM((chunk,), jnp.int32),
            pltpu.VMEM((chunk, D), jnp.float32),
            pltpu.VMEM((chunk, D), jnp.float32),
            pltpu.SemaphoreType.DMA,
            pltpu.SemaphoreType.DMA,
            pltpu.SemaphoreType.DMA,
            pltpu.SemaphoreType.DMA,
            pltpu.SemaphoreType.DMA,
            pltpu.SemaphoreType.DMA,
        ],
        compiler_params=pltpu.CompilerParams(use_tc_tiling_on_sc=False),
    )
    def k(idx_hbm, table_hbm, out_hbm, tab_v, i0, i1, r0, r1, si0, si1, sg0, sg1, sw0, sw1):
        wid = lax.axis_index("s") * NC + lax.axis_index("c")
        base = wid * b_per_w
        idx_v, rows_v = [i0, i1], [r0, r1]
        sem_i, sem_g, sem_w = [si0, si1], [sg0, sg1], [sw0, sw1]

        def start_idx(g, b):
            pltpu.make_async_copy(
                idx_hbm.at[pl.ds(base + g * chunk, chunk)], idx_v[b], sem_i[b]
            ).start()

        def wait_idx(b):
            pltpu.make_async_copy(
                idx_hbm.at[pl.ds(base, chunk)], idx_v[b], sem_i[b]
            ).wait()

        def start_gather(b):
            pltpu.make_async_copy(
                tab_v.at[idx_v[b]], rows_v[b], sem_g[b]
            ).start()

        def wait_gather(b):
            pltpu.make_async_copy(
                tab_v.at[idx_v[b]], rows_v[b], sem_g[b]
            ).wait()

        def start_wb(g, b):
            pltpu.make_async_copy(
                rows_v[b], out_hbm.at[pl.ds(base + g * chunk, chunk)], sem_w[b]
            ).start()

        def wait_wb(b):
            pltpu.make_async_copy(
                rows_v[b], out_hbm.at[pl.ds(base, chunk)], sem_w[b]
            ).wait()

        @pl.when(lax.axis_index("s") == 0)
        def _stage():
            pltpu.sync_copy(table_hbm, tab_v)

        plsc.subcore_barrier()

        for b in range(NBUF):
            start_idx(b, b)
        for b in range(NBUF):
            wait_idx(b)
            start_gather(b)

        def body(s, _):
            g0 = s * NBUF
            for b in range(NBUF):
                wait_gather(b)
                start_idx(g0 + NBUF + b, b)
                start_wb(g0 + b, b)
            for b in range(NBUF):
                wait_idx(b)
                wait_wb(b)
                start_gather(b)
            return 0

        lax.fori_loop(0, n_super - 1, body, 0)

        for b in range(NBUF):
            wait_gather(b)
            start_wb(n_chunks - NBUF + b, b)
        for b in range(NBUF):
            wait_wb(b)

    return k(edge_types, table)


def kernel(edge_types, table):
    return _sc_gather(edge_types.astype(jnp.int32), table, 1000)
